# Initial kernel scaffold; baseline (speedup 1.0000x reference)
import jax
import jax.numpy as jnp
from jax.experimental import pallas as pl


def kernel(x, edge_index, W, b):
    raise NotImplementedError("write your pallas kernel here")


# speedup vs baseline: 6.5118x; 6.5118x over previous
import functools

import jax
import jax.numpy as jnp
from jax import lax
from jax.experimental import pallas as pl
from jax.experimental.pallas import tpu as pltpu
from jax.experimental.pallas import tpu_sc as plsc

_N_PAD = 10240
_R = 320
_E = 320000
_E_PER_W = _E // 32
_SEQ = 500
_SEQ_P = 512
_D = 128
_NW = 32
_L = 16

_MESH = plsc.VectorSubcoreMesh(core_axis_name="c", subcore_axis_name="s")
_PARAMS = pltpu.CompilerParams(needs_layout_passes=False)


def _wid():
  return lax.axis_index("c") * 16 + lax.axis_index("s")


def _a_body(edge_hbm, flags_hbm, idx_v, flags_v, sem):
  w = _wid()
  zeros = jnp.zeros((_L,), jnp.int32)
  ones = jnp.ones((_L,), jnp.int32)

  def _zero(i, carry):
    flags_v[pl.ds(i * _L, _L)] = zeros
    return carry
  lax.fori_loop(0, _N_PAD // _L, _zero, 0, unroll=8)

  pltpu.sync_copy(edge_hbm.at[pl.ds(w * _E_PER_W, _E_PER_W)], idx_v)

  def _scatter(i, carry):
    ii = idx_v[pl.ds(i * _L, _L)]
    plsc.store_scatter(flags_v, [ii], ones)
    return carry
  lax.fori_loop(0, _E_PER_W // _L, _scatter, 0, unroll=8)

  pltpu.sync_copy(flags_v, flags_hbm.at[pl.ds(w * _N_PAD, _N_PAD)])


_kernel_a = functools.partial(
    pl.kernel,
    out_type=jax.ShapeDtypeStruct((_NW * _N_PAD,), jnp.int32),
    mesh=_MESH,
    compiler_params=_PARAMS,
    scratch_types=[
        pltpu.VMEM((_E_PER_W,), jnp.int32),
        pltpu.VMEM((_N_PAD,), jnp.int32),
        pltpu.SemaphoreType.DMA,
    ],
)(_a_body)


def _b_body(flags_hbm, comp_hbm, cnt_hbm, stage_v, comb_v, lcomp_v, cnt_v, sem):
  w = _wid()
  iota = lax.iota(jnp.int32, _L)

  for t in range(_NW):
    pltpu.sync_copy(flags_hbm.at[pl.ds(t * _N_PAD + w * _R, _R)],
                    stage_v.at[pl.ds(t * _R, _R)])

  def _combine(k, carry):
    v = stage_v[pl.ds(k * _L, _L)]
    for t in range(1, _NW):
      v = v | stage_v[pl.ds(t * _R + k * _L, _L)]
    comb_v[pl.ds(k * _L, _L)] = v
    return carry
  lax.fori_loop(0, _R // _L, _combine, 0, unroll=2)

  def _compact(k, carry):
    f = comb_v[pl.ds(k * _L, _L)]
    m = f > 0
    pos = carry + plsc.cumsum(f) - f
    vals = iota + (w * _R + k * _L)
    plsc.store_scatter(lcomp_v, [pos], vals, mask=m)
    return carry + jnp.sum(f)
  cnt = lax.fori_loop(0, _R // _L, _compact, jnp.int32(0), unroll=2)

  cnt_v[...] = jnp.full((_L,), cnt, jnp.int32)
  pltpu.sync_copy(cnt_v, cnt_hbm.at[pl.ds(w * _L, _L)])
  pltpu.sync_copy(lcomp_v, comp_hbm.at[pl.ds(w * _R, _R)])


_kernel_b = functools.partial(
    pl.kernel,
    out_type=(
        jax.ShapeDtypeStruct((_NW * _R,), jnp.int32),
        jax.ShapeDtypeStruct((_NW * _L,), jnp.int32),
    ),
    mesh=_MESH,
    compiler_params=_PARAMS,
    scratch_types=[
        pltpu.VMEM((_NW * _R,), jnp.int32),
        pltpu.VMEM((_R,), jnp.int32),
        pltpu.VMEM((_R,), jnp.int32),
        pltpu.VMEM((_L,), jnp.int32),
        pltpu.SemaphoreType.DMA,
    ],
)(_b_body)


def _c_body(comp_hbm, cnt_hbm, x_hbm, out_hbm,
            cntall_v, compall_v, nodes_v, rows_v, sem):
  w = _wid()
  iota = lax.iota(jnp.int32, _L)
  zeros = jnp.zeros((_L,), jnp.int32)

  pltpu.sync_copy(cnt_hbm, cntall_v)
  pltpu.sync_copy(comp_hbm, compall_v)

  cs = [cntall_v[pl.ds(t * _L, _L)][0] for t in range(_NW)]
  offs = []
  acc = jnp.int32(0)
  for t in range(_NW):
    offs.append(acc)
    acc = acc + cs[t]
  total = acc

  def _slot(jv):
    t_idx = zeros
    for t in range(_NW):
      t_idx = t_idx + (jv >= offs[t]).astype(jnp.int32)
    t_idx = t_idx - 1
    off_sel = zeros
    for t in range(_NW):
      off_sel = off_sel + (t_idx == t).astype(jnp.int32) * offs[t]
    loc = jv - off_sel
    valid = jv < total
    t_safe = jnp.where(valid, t_idx, zeros)
    l_safe = jnp.where(valid, loc, zeros)
    node = plsc.load_gather(compall_v, [t_safe * _R + l_safe])
    return node, valid

  node0, _ = _slot(zeros)
  jv = iota + w * _L
  n, v = _slot(jv)
  nodes_v[...] = jnp.where(v, n, node0)

  pltpu.async_copy(x_hbm.at[nodes_v], rows_v, sem).wait()
  pltpu.sync_copy(rows_v, out_hbm.at[pl.ds(w * _L, _L)])


_kernel_c = functools.partial(
    pl.kernel,
    out_type=jax.ShapeDtypeStruct((_SEQ_P, _D), jnp.float32),
    mesh=_MESH,
    compiler_params=_PARAMS,
    scratch_types=[
        pltpu.VMEM((_NW * _L,), jnp.int32),
        pltpu.VMEM((_NW * _R,), jnp.int32),
        pltpu.VMEM((_L,), jnp.int32),
        pltpu.VMEM((_L, _D), jnp.float32),
        pltpu.SemaphoreType.DMA,
    ],
)(_c_body)


def _mm_body(xg_ref, wt_ref, b_ref, o_ref):
  o_ref[...] = jnp.dot(xg_ref[...], wt_ref[...],
                       preferred_element_type=jnp.float32) + b_ref[...]


_project = pl.pallas_call(
    _mm_body,
    out_shape=jax.ShapeDtypeStruct((_SEQ_P, _D), jnp.float32),
)


@jax.jit
def kernel(x, edge_index, W, b):
  flags = _kernel_a(edge_index[0])
  comp, cnt = _kernel_b(flags)
  xg = _kernel_c(comp, cnt, x)
  out = _project(xg, W.T, b.reshape(1, _D))
  return out[None, :_SEQ, :]


# speedup vs baseline: 8.3514x; 1.2825x over previous
import functools

import jax
import jax.numpy as jnp
from jax import lax
from jax.experimental import pallas as pl
from jax.experimental.pallas import tpu as pltpu
from jax.experimental.pallas import tpu_sc as plsc

_N_PAD = 10240
_R = 320
_E = 320000
_E_PER_W = _E // 32
_SEQ = 500
_SEQ_P = 512
_D = 128
_NW = 32
_L = 16

_MESH = plsc.VectorSubcoreMesh(core_axis_name="c", subcore_axis_name="s")
_PARAMS = pltpu.CompilerParams(needs_layout_passes=False,
                               use_tc_tiling_on_sc=False)


def _wid():
  return lax.axis_index("c") * 16 + lax.axis_index("s")


def _a_body(edge_hbm, flags_hbm, idx_v, flags_v, sem):
  w = _wid()
  zeros = jnp.zeros((_L,), jnp.int32)
  ones = jnp.ones((_L,), jnp.int32)

  def _zero(i, carry):
    flags_v[pl.ds(i * _L, _L)] = zeros
    return carry
  lax.fori_loop(0, _N_PAD // _L, _zero, 0, unroll=8)

  pltpu.sync_copy(edge_hbm.at[pl.ds(w * _E_PER_W, _E_PER_W)], idx_v)

  def _scatter(i, carry):
    ii = idx_v[pl.ds(i * _L, _L)]
    plsc.store_scatter(flags_v, [ii], ones)
    return carry
  lax.fori_loop(0, _E_PER_W // _L, _scatter, 0, unroll=8)

  pltpu.sync_copy(flags_v, flags_hbm.at[w])


_kernel_a = functools.partial(
    pl.kernel,
    out_type=jax.ShapeDtypeStruct((_NW, _N_PAD), jnp.int32),
    mesh=_MESH,
    compiler_params=_PARAMS,
    scratch_types=[
        pltpu.VMEM((_E_PER_W,), jnp.int32),
        pltpu.VMEM((_N_PAD,), jnp.int32),
        pltpu.SemaphoreType.DMA,
    ],
)(_a_body)


def _b_body(flags_hbm, comp_hbm, cnt_hbm, stage_v, comb_v, lcomp_v, cnt_v, sem):
  w = _wid()
  iota = lax.iota(jnp.int32, _L)

  pltpu.sync_copy(flags_hbm.at[:, pl.ds(w * _R, _R)], stage_v)

  def _combine(k, carry):
    v = stage_v[0, pl.ds(k * _L, _L)]
    for t in range(1, _NW):
      v = v | stage_v[t, pl.ds(k * _L, _L)]
    comb_v[pl.ds(k * _L, _L)] = v
    return carry
  lax.fori_loop(0, _R // _L, _combine, 0, unroll=2)

  def _compact(k, carry):
    f = comb_v[pl.ds(k * _L, _L)]
    m = f > 0
    pos = carry + plsc.cumsum(f) - f
    vals = iota + (w * _R + k * _L)
    plsc.store_scatter(lcomp_v, [pos], vals, mask=m)
    return carry + jnp.sum(f)
  cnt = lax.fori_loop(0, _R // _L, _compact, jnp.int32(0), unroll=2)

  cnt_v[...] = jnp.full((_L,), cnt, jnp.int32)
  pltpu.sync_copy(cnt_v, cnt_hbm.at[pl.ds(w * _L, _L)])
  pltpu.sync_copy(lcomp_v, comp_hbm.at[pl.ds(w * _R, _R)])


_kernel_b = functools.partial(
    pl.kernel,
    out_type=(
        jax.ShapeDtypeStruct((_NW * _R,), jnp.int32),
        jax.ShapeDtypeStruct((_NW * _L,), jnp.int32),
    ),
    mesh=_MESH,
    compiler_params=_PARAMS,
    scratch_types=[
        pltpu.VMEM((_NW, _R), jnp.int32),
        pltpu.VMEM((_R,), jnp.int32),
        pltpu.VMEM((_R,), jnp.int32),
        pltpu.VMEM((_L,), jnp.int32),
        pltpu.SemaphoreType.DMA,
    ],
)(_b_body)


def _c_body(comp_hbm, cnt_hbm, x_hbm, out_hbm,
            cntall_v, compall_v, nodes_v, rows_v, sem):
  w = _wid()
  iota = lax.iota(jnp.int32, _L)
  zeros = jnp.zeros((_L,), jnp.int32)

  pltpu.sync_copy(cnt_hbm, cntall_v)
  pltpu.sync_copy(comp_hbm, compall_v)

  cs = [cntall_v[pl.ds(t * _L, _L)][0] for t in range(_NW)]
  offs = []
  acc = jnp.int32(0)
  for t in range(_NW):
    offs.append(acc)
    acc = acc + cs[t]
  total = acc

  def _slot(jv):
    t_idx = zeros
    for t in range(_NW):
      t_idx = t_idx + (jv >= offs[t]).astype(jnp.int32)
    t_idx = t_idx - 1
    off_sel = zeros
    for t in range(_NW):
      off_sel = off_sel + (t_idx == t).astype(jnp.int32) * offs[t]
    loc = jv - off_sel
    valid = jv < total
    t_safe = jnp.where(valid, t_idx, zeros)
    l_safe = jnp.where(valid, loc, zeros)
    node = plsc.load_gather(compall_v, [t_safe * _R + l_safe])
    return node, valid

  node0, _ = _slot(zeros)
  jv = iota + w * _L
  n, v = _slot(jv)
  nodes_v[...] = jnp.where(v, n, node0)

  pltpu.async_copy(x_hbm.at[nodes_v], rows_v, sem).wait()
  pltpu.sync_copy(rows_v, out_hbm.at[pl.ds(w * _L, _L)])


_kernel_c = functools.partial(
    pl.kernel,
    out_type=jax.ShapeDtypeStruct((_SEQ_P, _D), jnp.float32),
    mesh=_MESH,
    compiler_params=_PARAMS,
    scratch_types=[
        pltpu.VMEM((_NW * _L,), jnp.int32),
        pltpu.VMEM((_NW * _R,), jnp.int32),
        pltpu.VMEM((_L,), jnp.int32),
        pltpu.VMEM((_L, _D), jnp.float32),
        pltpu.SemaphoreType.DMA,
    ],
)(_c_body)


def _mm_body(xg_ref, wt_ref, b_ref, o_ref):
  o_ref[...] = jnp.dot(xg_ref[...], wt_ref[...],
                       preferred_element_type=jnp.float32) + b_ref[...]


_project = pl.pallas_call(
    _mm_body,
    out_shape=jax.ShapeDtypeStruct((_SEQ_P, _D), jnp.float32),
)


@jax.jit
def kernel(x, edge_index, W, b):
  flags = _kernel_a(edge_index[0])
  comp, cnt = _kernel_b(flags)
  xg = _kernel_c(comp, cnt, x)
  out = _project(xg, W.T, b.reshape(1, _D))
  return out[None, :_SEQ, :]


# speedup vs baseline: 8.4999x; 1.0178x over previous
import functools

import jax
import jax.numpy as jnp
from jax import lax
from jax.experimental import pallas as pl
from jax.experimental.pallas import tpu as pltpu
from jax.experimental.pallas import tpu_sc as plsc

_N_PAD = 10240
_R = 320
_E = 320000
_E_PER_W = _E // 32
_SEQ = 500
_SEQ_P = 512
_D = 128
_NW = 32
_L = 16

_MESH = plsc.VectorSubcoreMesh(core_axis_name="c", subcore_axis_name="s")
_PARAMS = pltpu.CompilerParams(needs_layout_passes=False,
                               use_tc_tiling_on_sc=False)


def _wid():
  return lax.axis_index("c") * 16 + lax.axis_index("s")


def _a_body(edge_hbm, flags_hbm, idx_v, flags_v, sem):
  w = _wid()
  zeros = jnp.zeros((_L,), jnp.int32)
  ones = jnp.ones((_L,), jnp.int32)

  def _zero(i, carry):
    flags_v[pl.ds(i * _L, _L)] = zeros
    return carry
  lax.fori_loop(0, _N_PAD // _L, _zero, 0, unroll=8)

  pltpu.sync_copy(edge_hbm.at[pl.ds(w * _E_PER_W, _E_PER_W)], idx_v)

  def _scatter(i, carry):
    ii = idx_v[pl.ds(i * _L, _L)]
    plsc.store_scatter(flags_v, [ii], ones)
    return carry
  lax.fori_loop(0, _E_PER_W // _L, _scatter, 0, unroll=8)

  pltpu.sync_copy(flags_v, flags_hbm.at[w])


_kernel_a = functools.partial(
    pl.kernel,
    out_type=jax.ShapeDtypeStruct((_NW, _N_PAD), jnp.int32),
    mesh=_MESH,
    compiler_params=_PARAMS,
    scratch_types=[
        pltpu.VMEM((_E_PER_W,), jnp.int32),
        pltpu.VMEM((_N_PAD,), jnp.int32),
        pltpu.SemaphoreType.DMA,
    ],
)(_a_body)


def _b_body(flags_hbm, comp_hbm, cnt_hbm, stage_v, comb_v, lcomp_v, cnt_v, sem):
  w = _wid()
  iota = lax.iota(jnp.int32, _L)

  pltpu.sync_copy(flags_hbm.at[:, pl.ds(w * _R, _R)], stage_v)

  def _combine(k, carry):
    v = stage_v[0, pl.ds(k * _L, _L)]
    for t in range(1, _NW):
      v = v | stage_v[t, pl.ds(k * _L, _L)]
    comb_v[pl.ds(k * _L, _L)] = v
    return carry
  lax.fori_loop(0, _R // _L, _combine, 0, unroll=2)

  def _compact(k, carry):
    f = comb_v[pl.ds(k * _L, _L)]
    m = f > 0
    pos = carry + plsc.cumsum(f) - f
    vals = iota + (w * _R + k * _L)
    plsc.store_scatter(lcomp_v, [pos], vals, mask=m)
    return carry + jnp.sum(f)
  cnt = lax.fori_loop(0, _R // _L, _compact, jnp.int32(0), unroll=2)

  cnt_v[...] = jnp.full((_L,), cnt, jnp.int32)
  pltpu.sync_copy(cnt_v, cnt_hbm.at[pl.ds(w * _L, _L)])
  pltpu.sync_copy(lcomp_v, comp_hbm.at[pl.ds(w * _R, _R)])


_kernel_b = functools.partial(
    pl.kernel,
    out_type=(
        jax.ShapeDtypeStruct((_NW * _R,), jnp.int32),
        jax.ShapeDtypeStruct((_NW * _L,), jnp.int32),
    ),
    mesh=_MESH,
    compiler_params=_PARAMS,
    scratch_types=[
        pltpu.VMEM((_NW, _R), jnp.int32),
        pltpu.VMEM((_R,), jnp.int32),
        pltpu.VMEM((_R,), jnp.int32),
        pltpu.VMEM((_L,), jnp.int32),
        pltpu.SemaphoreType.DMA,
    ],
)(_b_body)


def _c_body(comp_hbm, cnt_hbm, x_hbm, out_hbm,
            cntall_v, compall_v, nodes_v, rows_v, sem):
  w = _wid()
  iota = lax.iota(jnp.int32, _L)
  zeros = jnp.zeros((_L,), jnp.int32)

  pltpu.sync_copy(cnt_hbm, cntall_v)
  pltpu.sync_copy(comp_hbm, compall_v)

  cs = [cntall_v[pl.ds(t * _L, _L)][0] for t in range(_NW)]
  offs = []
  acc = jnp.int32(0)
  for t in range(_NW):
    offs.append(acc)
    acc = acc + cs[t]
  total = acc

  def _slot(jv):
    t_idx = zeros
    for t in range(_NW):
      t_idx = t_idx + (jv >= offs[t]).astype(jnp.int32)
    t_idx = t_idx - 1
    off_sel = zeros
    for t in range(_NW):
      off_sel = off_sel + (t_idx == t).astype(jnp.int32) * offs[t]
    loc = jv - off_sel
    valid = jv < total
    t_safe = jnp.where(valid, t_idx, zeros)
    l_safe = jnp.where(valid, loc, zeros)
    node = plsc.load_gather(compall_v, [t_safe * _R + l_safe])
    return node, valid

  node0, _ = _slot(zeros)
  jv = iota + w * _L
  n, v = _slot(jv)
  nodes_v[...] = jnp.where(v, n, node0)

  pltpu.async_copy(x_hbm.at[nodes_v], rows_v, sem).wait()
  pltpu.sync_copy(rows_v, out_hbm.at[pl.ds(w * _L, _L)])


_kernel_c = functools.partial(
    pl.kernel,
    out_type=jax.ShapeDtypeStruct((_SEQ_P, _D), jnp.float32),
    mesh=_MESH,
    compiler_params=_PARAMS,
    scratch_types=[
        pltpu.VMEM((_NW * _L,), jnp.int32),
        pltpu.VMEM((_NW * _R,), jnp.int32),
        pltpu.VMEM((_L,), jnp.int32),
        pltpu.VMEM((_L, _D), jnp.float32),
        pltpu.SemaphoreType.DMA,
    ],
)(_c_body)


def _mm_body(x_ref, wt_ref, b_ref, o_ref):
  o_ref[...] = jnp.dot(x_ref[...], wt_ref[...],
                       preferred_element_type=jnp.float32) + b_ref[...]


_project = pl.pallas_call(
    _mm_body,
    out_shape=jax.ShapeDtypeStruct((10000, _D), jnp.float32),
)


@jax.jit
def kernel(x, edge_index, W, b):
  proj = _project(x, W.T, b.reshape(1, _D))
  flags = _kernel_a(edge_index[0])
  comp, cnt = _kernel_b(flags)
  out = _kernel_c(comp, cnt, proj)
  return out[None, :_SEQ, :]


# speedup vs baseline: 8.6044x; 1.0123x over previous
import functools

import jax
import jax.numpy as jnp
from jax import lax
from jax.experimental import pallas as pl
from jax.experimental.pallas import tpu as pltpu
from jax.experimental.pallas import tpu_sc as plsc

_N = 10000
_N_PAD = 10240
_E = 320000
_E_PER_W = _E // 32
_SEQ = 500
_SEQ_P = 512
_D = 128
_NW = 32
_L = 16
_NR = 20
_RR = 512

_MESH = plsc.VectorSubcoreMesh(core_axis_name="c", subcore_axis_name="s")
_PARAMS = pltpu.CompilerParams(needs_layout_passes=False,
                               use_tc_tiling_on_sc=False)


def _wid():
  return lax.axis_index("c") * 16 + lax.axis_index("s")


def _a_body(edge_hbm, flags_hbm, idx_v, flags_v, sem):
  w = _wid()
  zeros = jnp.zeros((_L,), jnp.int32)
  ones = jnp.ones((_L,), jnp.int32)

  def _zero(i, carry):
    flags_v[pl.ds(i * _L, _L)] = zeros
    return carry
  lax.fori_loop(0, _N_PAD // _L, _zero, 0, unroll=8)

  pltpu.sync_copy(edge_hbm.at[pl.ds(w * _E_PER_W, _E_PER_W)], idx_v)

  def _scatter(i, carry):
    ii = idx_v[pl.ds(i * _L, _L)]
    plsc.store_scatter(flags_v, [ii], ones)
    return carry
  lax.fori_loop(0, _E_PER_W // _L, _scatter, 0, unroll=8)

  pltpu.sync_copy(flags_v, flags_hbm.at[w])


_kernel_a = functools.partial(
    pl.kernel,
    out_type=jax.ShapeDtypeStruct((_NW, _N_PAD), jnp.int32),
    mesh=_MESH,
    compiler_params=_PARAMS,
    scratch_types=[
        pltpu.VMEM((_E_PER_W,), jnp.int32),
        pltpu.VMEM((_N_PAD,), jnp.int32),
        pltpu.SemaphoreType.DMA,
    ],
)(_a_body)


def _or_body(f_ref, o_ref):
  acc = f_ref[0]
  for t in range(1, _NW):
    acc = acc | f_ref[t]
  o_ref[...] = acc


_combine = pl.pallas_call(
    _or_body,
    out_shape=jax.ShapeDtypeStruct((_N_PAD // _D, _D), jnp.int32),
)


def _c_body(comb_hbm, proj_hbm, out_hbm, comb_v, lcomp_v, lcomp0_v,
            nodes_v, rows_v, sem):
  w = _wid()
  iota = lax.iota(jnp.int32, _L)
  zeros = jnp.zeros((_L,), jnp.int32)

  pltpu.sync_copy(comb_hbm, comb_v)

  cs = []
  for r in range(_NR):
    def _acc(g, carry):
      return carry + comb_v[pl.ds(r * _RR + g * _L, _L)]
    acc = lax.fori_loop(0, _RR // _L, _acc, zeros, unroll=4)
    cs.append(jnp.sum(acc))
  offs = []
  tot = jnp.int32(0)
  for r in range(_NR):
    offs.append(tot)
    tot = tot + cs[r]
  total = tot

  j_lo = jnp.int32(w * _L)
  j_hi = jnp.minimum(j_lo + _L - 1, jnp.maximum(total - 1, 0))
  j_lo_c = jnp.minimum(j_lo, jnp.maximum(total - 1, 0))
  t_lo = jnp.int32(0)
  t_hi = jnp.int32(0)
  t0 = jnp.int32(0)
  off_lo = jnp.int32(0)
  for r in range(_NR):
    t_lo = t_lo + (offs[r] <= j_lo_c).astype(jnp.int32)
    t_hi = t_hi + (offs[r] <= j_hi).astype(jnp.int32)
    t0 = t0 + (offs[r] <= 0).astype(jnp.int32)
  t_lo = t_lo - 1
  t_hi = jnp.maximum(t_hi - 1, t_lo)
  t0 = t0 - 1
  for r in range(_NR):
    off_lo = off_lo + jnp.where(r < t_lo, cs[r], 0)

  def _compact_ranges(r_start, r_end, out_ref):
    def _outer(r, carry):
      def _inner(g, c2):
        f = comb_v[pl.ds(r * _RR + g * _L, _L)]
        m = f > 0
        pos = c2 + plsc.cumsum(f) - f
        vals = iota + (r * _RR + g * _L)
        plsc.store_scatter(out_ref, [pos], vals, mask=m)
        return c2 + jnp.sum(f)
      return lax.fori_loop(0, _RR // _L, _inner, carry)
    return lax.fori_loop(r_start, r_end, _outer, jnp.int32(0))

  _compact_ranges(t_lo, t_hi + 1, lcomp_v)
  _compact_ranges(t0, t0 + 1, lcomp0_v)
  node0 = lcomp0_v[pl.ds(0, _L)][0]

  jv = iota + j_lo
  valid = jv < total
  lidx = jnp.where(valid, jv - off_lo, zeros)
  node = plsc.load_gather(lcomp_v, [lidx])
  nodes_v[...] = jnp.where(valid, node, jnp.full((_L,), node0, jnp.int32))

  pltpu.async_copy(proj_hbm.at[nodes_v], rows_v, sem).wait()
  pltpu.sync_copy(rows_v, out_hbm.at[pl.ds(w * _L, _L)])


_kernel_c = functools.partial(
    pl.kernel,
    out_type=jax.ShapeDtypeStruct((_SEQ_P, _D), jnp.float32),
    mesh=_MESH,
    compiler_params=_PARAMS,
    scratch_types=[
        pltpu.VMEM((_N_PAD,), jnp.int32),
        pltpu.VMEM((2048,), jnp.int32),
        pltpu.VMEM((_RR,), jnp.int32),
        pltpu.VMEM((_L,), jnp.int32),
        pltpu.VMEM((_L, _D), jnp.float32),
        pltpu.SemaphoreType.DMA,
    ],
)(_c_body)


def _mm_body(x_ref, wt_ref, b_ref, o_ref):
  o_ref[...] = jnp.dot(x_ref[...], wt_ref[...],
                       preferred_element_type=jnp.float32) + b_ref[...]


_project = pl.pallas_call(
    _mm_body,
    out_shape=jax.ShapeDtypeStruct((_N, _D), jnp.float32),
)


@jax.jit
def kernel(x, edge_index, W, b):
  proj = _project(x, W.T, b.reshape(1, _D))
  flags = _kernel_a(edge_index[0])
  comb = _combine(flags.reshape(_NW, _N_PAD // _D, _D))
  out = _kernel_c(comb.reshape(_N_PAD), proj)
  return out[None, :_SEQ, :]
